# baseline (speedup 1.0000x reference)
import functools

import jax
import jax.numpy as jnp
from jax import lax
from jax.experimental import pallas as pl
from jax.experimental.pallas import tpu as pltpu
from jax.experimental.pallas import tpu_sc as plsc

N_NODES = 10000
E_EDGES = 160000
D = 32
R = 16

LANE = 128
NROWS = 1280
PAD_E = NROWS * LANE
N_CORES = 1
NTILES = 16 * N_CORES
ROWS_PER_TILE = NROWS // NTILES
ACC_ROWS = 10240
STRIPE = ACC_ROWS // 16


def _mm_kernel(feat_ref, w2_ref, hid_ref):
    hid_ref[0] = jnp.dot(feat_ref[...], w2_ref[...],
                         preferred_element_type=jnp.float32)


def _gidx_kernel(src_ref, et_ref, gidx_ref):
    et = et_ref[...]
    gidx_ref[...] = (et >> 2) * (4 * N_NODES) + src_ref[...] * 4 + (et & 3)


BANK = 8
NBATCH = ROWS_PER_TILE // BANK


def _sc_body(gidx_hbm, dst_hbm, hidden_hbm, zeros_hbm, out_hbm,
             gidx_v, dst_v, msg_v, acc_sh, sem0, sem1, ssem0, ssem1):
    ssems = (ssem0, ssem1)
    c = lax.axis_index("c")
    s = lax.axis_index("s")
    wid = s * N_CORES + c

    pltpu.sync_copy(zeros_hbm.at[pl.ds(s * STRIPE, STRIPE)],
                    acc_sh.at[pl.ds(s * STRIPE, STRIPE)])
    plsc.subcore_barrier()

    row0 = wid * ROWS_PER_TILE
    pltpu.sync_copy(gidx_hbm.at[pl.ds(row0, ROWS_PER_TILE)], gidx_v)
    pltpu.sync_copy(dst_hbm.at[pl.ds(row0, ROWS_PER_TILE)], dst_v)

    gsems = (sem0, sem1)
    gds = {}
    sds = {}

    def gstart(t):
        b0 = (t % 2) * BANK
        gds[t] = [
            pltpu.async_copy(hidden_hbm.at[gidx_v.at[t * BANK + i]],
                             msg_v.at[b0 + i], gsems[t % 2])
            for i in range(BANK)
        ]

    def sstart(t):
        b0 = (t % 2) * BANK
        sds[t] = [
            pltpu.async_copy(msg_v.at[b0 + i],
                             acc_sh.at[dst_v.at[t * BANK + i]],
                             ssems[t % 2], add=True)
            for i in range(BANK)
        ]

    gstart(0)
    for t in range(NBATCH):
        if t + 1 < NBATCH:
            if t >= 1:
                for d in sds[t - 1]:
                    d.wait()
            gstart(t + 1)
        for d in gds[t]:
            d.wait()
        sstart(t)
    for d in sds[NBATCH - 2] + sds[NBATCH - 1]:
        d.wait()

    plsc.subcore_barrier()

    pltpu.sync_copy(acc_sh.at[pl.ds(s * STRIPE, STRIPE)],
                    out_hbm.at[c, pl.ds(s * STRIPE, STRIPE)])


def _add_kernel(p_ref, o_ref):
    p = p_ref[...]
    o_ref[...] = p[0, :N_NODES] + p[1, :N_NODES]


@jax.jit
def kernel(feat, edge_index, etypes, weight):
    src = edge_index[0]
    dst = edge_index[1]
    w2 = weight.transpose(1, 0, 2).reshape(D, R * D)
    pad = PAD_E - E_EDGES
    src2d = jnp.pad(src, (0, pad)).reshape(NROWS, LANE)
    et2d = jnp.pad(etypes, (0, pad)).reshape(NROWS, LANE)
    dst2d = jnp.pad(dst, (0, pad), constant_values=N_NODES).reshape(NROWS, LANE)
    zeros = jnp.zeros((ACC_ROWS, D), jnp.float32)

    hidden = pl.pallas_call(
        _mm_kernel,
        grid=(1, 4),
        in_specs=[
            pl.BlockSpec((N_NODES, D), lambda i, j: (0, 0)),
            pl.BlockSpec((D, LANE), lambda i, j: (0, j)),
        ],
        out_specs=pl.BlockSpec((1, N_NODES, LANE), lambda i, j: (j, 0, 0)),
        out_shape=jax.ShapeDtypeStruct((4, N_NODES, LANE), jnp.float32),
    )(feat, w2)
    hidden = hidden.reshape(N_NODES * R, D)
    gidx2d = pl.pallas_call(
        _gidx_kernel,
        out_shape=jax.ShapeDtypeStruct((NROWS, LANE), jnp.int32),
    )(src2d, et2d)

    mesh = plsc.VectorSubcoreMesh(core_axis_name="c", subcore_axis_name="s",
                                  num_cores=N_CORES)
    partials = pl.kernel(
        _sc_body,
        out_type=jax.ShapeDtypeStruct((N_CORES, ACC_ROWS, D), jnp.float32),
        mesh=mesh,
        scratch_types=[
            pltpu.VMEM((ROWS_PER_TILE, LANE), jnp.int32),
            pltpu.VMEM((ROWS_PER_TILE, LANE), jnp.int32),
            pltpu.VMEM((2 * BANK, LANE, D), jnp.float32),
            pltpu.VMEM_SHARED((ACC_ROWS, D), jnp.float32),
            pltpu.SemaphoreType.DMA,
            pltpu.SemaphoreType.DMA,
            pltpu.SemaphoreType.DMA,
            pltpu.SemaphoreType.DMA,
        ],
        compiler_params=pltpu.CompilerParams(use_tc_tiling_on_sc=False),
    )(gidx2d, dst2d, hidden, zeros)

    if N_CORES == 1:
        return partials[0, :N_NODES]
    out = pl.pallas_call(
        _add_kernel,
        out_shape=jax.ShapeDtypeStruct((N_NODES, D), jnp.float32),
    )(partials)
    return out


# speedup vs baseline: 1.0031x; 1.0031x over previous
import functools

import jax
import jax.numpy as jnp
from jax import lax
from jax.experimental import pallas as pl
from jax.experimental.pallas import tpu as pltpu
from jax.experimental.pallas import tpu_sc as plsc

N_NODES = 10000
E_EDGES = 160000
D = 32
R = 16

LANE = 128
NROWS = 1280
PAD_E = NROWS * LANE
N_CORES = 1
NTILES = 16 * N_CORES
ROWS_PER_TILE = NROWS // NTILES
ACC_ROWS = 10240
STRIPE = ACC_ROWS // 16


def _mm_kernel(feat_ref, w2_ref, hid_ref):
    hid_ref[0] = jnp.dot(feat_ref[...], w2_ref[...],
                         preferred_element_type=jnp.float32)


def _gidx_kernel(src_ref, et_ref, gidx_ref):
    et = et_ref[...]
    gidx_ref[...] = (et >> 2) * (4 * N_NODES) + src_ref[...] * 4 + (et & 3)


BANK = 10
NBATCH = ROWS_PER_TILE // BANK


def _sc_body(gidx_hbm, dst_hbm, hidden_hbm, zeros_hbm, out_hbm,
             gidx_v, dst_v, msg_v, acc_sh, sem0, sem1, ssem0, ssem1):
    ssems = (ssem0, ssem1)
    c = lax.axis_index("c")
    s = lax.axis_index("s")
    wid = s * N_CORES + c

    pltpu.sync_copy(zeros_hbm.at[pl.ds(s * STRIPE, STRIPE)],
                    acc_sh.at[pl.ds(s * STRIPE, STRIPE)])
    plsc.subcore_barrier()

    row0 = wid * ROWS_PER_TILE
    pltpu.sync_copy(gidx_hbm.at[pl.ds(row0, ROWS_PER_TILE)], gidx_v)
    pltpu.sync_copy(dst_hbm.at[pl.ds(row0, ROWS_PER_TILE)], dst_v)

    gsems = (sem0, sem1)
    gds = {}
    sds = {}

    def gstart(t):
        b0 = (t % 2) * BANK
        gds[t] = [
            pltpu.async_copy(hidden_hbm.at[gidx_v.at[t * BANK + i]],
                             msg_v.at[b0 + i], gsems[t % 2])
            for i in range(BANK)
        ]

    def sstart(t):
        b0 = (t % 2) * BANK
        sds[t] = [
            pltpu.async_copy(msg_v.at[b0 + i],
                             acc_sh.at[dst_v.at[t * BANK + i]],
                             ssems[t % 2], add=True)
            for i in range(BANK)
        ]

    gstart(0)
    for t in range(NBATCH):
        if t + 1 < NBATCH:
            if t >= 1:
                for d in sds[t - 1]:
                    d.wait()
            gstart(t + 1)
        for d in gds[t]:
            d.wait()
        sstart(t)
    for d in sds[NBATCH - 2] + sds[NBATCH - 1]:
        d.wait()

    plsc.subcore_barrier()

    pltpu.sync_copy(acc_sh.at[pl.ds(s * STRIPE, STRIPE)],
                    out_hbm.at[c, pl.ds(s * STRIPE, STRIPE)])


def _add_kernel(p_ref, o_ref):
    p = p_ref[...]
    o_ref[...] = p[0, :N_NODES] + p[1, :N_NODES]


@jax.jit
def kernel(feat, edge_index, etypes, weight):
    src = edge_index[0]
    dst = edge_index[1]
    w2 = weight.transpose(1, 0, 2).reshape(D, R * D)
    pad = PAD_E - E_EDGES
    src2d = jnp.pad(src, (0, pad)).reshape(NROWS, LANE)
    et2d = jnp.pad(etypes, (0, pad)).reshape(NROWS, LANE)
    dst2d = jnp.pad(dst, (0, pad), constant_values=N_NODES).reshape(NROWS, LANE)
    zeros = jnp.zeros((ACC_ROWS, D), jnp.float32)

    hidden = pl.pallas_call(
        _mm_kernel,
        grid=(1, 4),
        in_specs=[
            pl.BlockSpec((N_NODES, D), lambda i, j: (0, 0)),
            pl.BlockSpec((D, LANE), lambda i, j: (0, j)),
        ],
        out_specs=pl.BlockSpec((1, N_NODES, LANE), lambda i, j: (j, 0, 0)),
        out_shape=jax.ShapeDtypeStruct((4, N_NODES, LANE), jnp.float32),
    )(feat, w2)
    hidden = hidden.reshape(N_NODES * R, D)
    gidx2d = pl.pallas_call(
        _gidx_kernel,
        out_shape=jax.ShapeDtypeStruct((NROWS, LANE), jnp.int32),
    )(src2d, et2d)

    mesh = plsc.VectorSubcoreMesh(core_axis_name="c", subcore_axis_name="s",
                                  num_cores=N_CORES)
    partials = pl.kernel(
        _sc_body,
        out_type=jax.ShapeDtypeStruct((N_CORES, ACC_ROWS, D), jnp.float32),
        mesh=mesh,
        scratch_types=[
            pltpu.VMEM((ROWS_PER_TILE, LANE), jnp.int32),
            pltpu.VMEM((ROWS_PER_TILE, LANE), jnp.int32),
            pltpu.VMEM((2 * BANK, LANE, D), jnp.float32),
            pltpu.VMEM_SHARED((ACC_ROWS, D), jnp.float32),
            pltpu.SemaphoreType.DMA,
            pltpu.SemaphoreType.DMA,
            pltpu.SemaphoreType.DMA,
            pltpu.SemaphoreType.DMA,
        ],
        compiler_params=pltpu.CompilerParams(use_tc_tiling_on_sc=False),
    )(gidx2d, dst2d, hidden, zeros)

    if N_CORES == 1:
        return partials[0, :N_NODES]
    out = pl.pallas_call(
        _add_kernel,
        out_shape=jax.ShapeDtypeStruct((N_NODES, D), jnp.float32),
    )(partials)
    return out


# speedup vs baseline: 1.0967x; 1.0933x over previous
import functools

import jax
import jax.numpy as jnp
from jax import lax
from jax.experimental import pallas as pl
from jax.experimental.pallas import tpu as pltpu
from jax.experimental.pallas import tpu_sc as plsc

N_NODES = 10000
E_EDGES = 160000
D = 32
R = 16

LANE = 128
NROWS = 1280
PAD_E = NROWS * LANE
N_CORES = 1
NTILES = 16 * N_CORES
ROWS_PER_TILE = NROWS // NTILES
ACC_ROWS = 10240
STRIPE = ACC_ROWS // 16


def _mm_kernel(feat_ref, w2_ref, hid_ref):
    hid_ref[0] = jnp.dot(feat_ref[...], w2_ref[...],
                         preferred_element_type=jnp.float32)


E_ROWS = E_EDGES // LANE
E_ALIGNED = 1248


def _prep_kernel(src_ref, et_ref, dst_ref, gidx_ref, dst2_ref):
    et = et_ref[...]
    g = (et >> 2) * (4 * N_NODES) + src_ref[...] * 4 + (et & 3)
    d = dst_ref[...]
    gidx_ref[0:E_ALIGNED] = g[0:E_ALIGNED]
    dst2_ref[0:E_ALIGNED] = d[0:E_ALIGNED]
    gidx_ref[E_ALIGNED:NROWS] = jnp.concatenate(
        [g[E_ALIGNED:E_ROWS],
         jnp.zeros((NROWS - E_ROWS, LANE), jnp.int32)], axis=0)
    dst2_ref[E_ALIGNED:NROWS] = jnp.concatenate(
        [d[E_ALIGNED:E_ROWS],
         jnp.full((NROWS - E_ROWS, LANE), N_NODES, jnp.int32)], axis=0)


BANK = 10
NBATCH = ROWS_PER_TILE // BANK


def _sc_body(gidx_hbm, dst_hbm, hidden_hbm, zeros_hbm, out_hbm,
             gidx_v, dst_v, msg_v, acc_sh, sem0, sem1, ssem0, ssem1):
    ssems = (ssem0, ssem1)
    c = lax.axis_index("c")
    s = lax.axis_index("s")
    wid = s * N_CORES + c

    pltpu.sync_copy(zeros_hbm.at[pl.ds(s * STRIPE, STRIPE)],
                    acc_sh.at[pl.ds(s * STRIPE, STRIPE)])
    plsc.subcore_barrier()

    row0 = wid * ROWS_PER_TILE
    pltpu.sync_copy(gidx_hbm.at[pl.ds(row0, ROWS_PER_TILE)], gidx_v)
    pltpu.sync_copy(dst_hbm.at[pl.ds(row0, ROWS_PER_TILE)], dst_v)

    gsems = (sem0, sem1)
    gds = {}
    sds = {}

    def gstart(t):
        b0 = (t % 2) * BANK
        gds[t] = [
            pltpu.async_copy(hidden_hbm.at[gidx_v.at[t * BANK + i]],
                             msg_v.at[b0 + i], gsems[t % 2])
            for i in range(BANK)
        ]

    def sstart(t):
        b0 = (t % 2) * BANK
        sds[t] = [
            pltpu.async_copy(msg_v.at[b0 + i],
                             acc_sh.at[dst_v.at[t * BANK + i]],
                             ssems[t % 2], add=True)
            for i in range(BANK)
        ]

    gstart(0)
    for t in range(NBATCH):
        if t + 1 < NBATCH:
            if t >= 1:
                for d in sds[t - 1]:
                    d.wait()
            gstart(t + 1)
        for d in gds[t]:
            d.wait()
        sstart(t)
    for d in sds[NBATCH - 2] + sds[NBATCH - 1]:
        d.wait()

    plsc.subcore_barrier()

    pltpu.sync_copy(acc_sh.at[pl.ds(s * STRIPE, STRIPE)],
                    out_hbm.at[c, pl.ds(s * STRIPE, STRIPE)])


def _add_kernel(p_ref, o_ref):
    p = p_ref[...]
    o_ref[...] = p[0, :N_NODES] + p[1, :N_NODES]


@jax.jit
def kernel(feat, edge_index, etypes, weight):
    src = edge_index[0]
    dst = edge_index[1]
    w2 = weight.transpose(1, 0, 2).reshape(D, R * D)
    src_r = src.reshape(E_ROWS, LANE)
    et_r = etypes.reshape(E_ROWS, LANE)
    dst_r = dst.reshape(E_ROWS, LANE)
    zeros = jnp.zeros((ACC_ROWS, D), jnp.float32)

    hidden = pl.pallas_call(
        _mm_kernel,
        grid=(1, 4),
        in_specs=[
            pl.BlockSpec((N_NODES, D), lambda i, j: (0, 0)),
            pl.BlockSpec((D, LANE), lambda i, j: (0, j)),
        ],
        out_specs=pl.BlockSpec((1, N_NODES, LANE), lambda i, j: (j, 0, 0)),
        out_shape=jax.ShapeDtypeStruct((4, N_NODES, LANE), jnp.float32),
    )(feat, w2)
    hidden = hidden.reshape(N_NODES * R, D)
    gidx2d, dst2d = pl.pallas_call(
        _prep_kernel,
        out_shape=[
            jax.ShapeDtypeStruct((NROWS, LANE), jnp.int32),
            jax.ShapeDtypeStruct((NROWS, LANE), jnp.int32),
        ],
    )(src_r, et_r, dst_r)

    mesh = plsc.VectorSubcoreMesh(core_axis_name="c", subcore_axis_name="s",
                                  num_cores=N_CORES)
    partials = pl.kernel(
        _sc_body,
        out_type=jax.ShapeDtypeStruct((N_CORES, ACC_ROWS, D), jnp.float32),
        mesh=mesh,
        scratch_types=[
            pltpu.VMEM((ROWS_PER_TILE, LANE), jnp.int32),
            pltpu.VMEM((ROWS_PER_TILE, LANE), jnp.int32),
            pltpu.VMEM((2 * BANK, LANE, D), jnp.float32),
            pltpu.VMEM_SHARED((ACC_ROWS, D), jnp.float32),
            pltpu.SemaphoreType.DMA,
            pltpu.SemaphoreType.DMA,
            pltpu.SemaphoreType.DMA,
            pltpu.SemaphoreType.DMA,
        ],
        compiler_params=pltpu.CompilerParams(use_tc_tiling_on_sc=False),
    )(gidx2d, dst2d, hidden, zeros)

    if N_CORES == 1:
        return partials[0, :N_NODES]
    out = pl.pallas_call(
        _add_kernel,
        out_shape=jax.ShapeDtypeStruct((N_NODES, D), jnp.float32),
    )(partials)
    return out


# speedup vs baseline: 1.0968x; 1.0001x over previous
import jax
import jax.numpy as jnp
from jax import lax
from jax.experimental import pallas as pl
from jax.experimental.pallas import tpu as pltpu
from jax.experimental.pallas import tpu_sc as plsc

N_NODES = 10000
E_EDGES = 160000
D = 32
R = 16

LANE = 128
NROWS = 1280
PAD_E = NROWS * LANE
N_CORES = 1
NTILES = 16 * N_CORES
ROWS_PER_TILE = NROWS // NTILES
ACC_ROWS = 10240
STRIPE = ACC_ROWS // 16


def _mm_kernel(feat_ref, w2_ref, hid_ref):
    hid_ref[0] = jnp.dot(feat_ref[...], w2_ref[...],
                         preferred_element_type=jnp.float32)


E_ROWS = E_EDGES // LANE
E_ALIGNED = 1248


def _prep_kernel(src_ref, et_ref, dst_ref, gidx_ref, dst2_ref):
    et = et_ref[...]
    g = (et >> 2) * (4 * N_NODES) + src_ref[...] * 4 + (et & 3)
    d = dst_ref[...]
    gidx_ref[0:E_ALIGNED] = g[0:E_ALIGNED]
    dst2_ref[0:E_ALIGNED] = d[0:E_ALIGNED]
    gidx_ref[E_ALIGNED:NROWS] = jnp.concatenate(
        [g[E_ALIGNED:E_ROWS],
         jnp.zeros((NROWS - E_ROWS, LANE), jnp.int32)], axis=0)
    dst2_ref[E_ALIGNED:NROWS] = jnp.concatenate(
        [d[E_ALIGNED:E_ROWS],
         jnp.full((NROWS - E_ROWS, LANE), N_NODES, jnp.int32)], axis=0)


BANK = 10
NBATCH = ROWS_PER_TILE // BANK


def _sc_body(gidx_hbm, dst_hbm, hidden_hbm, zeros_hbm, out_hbm,
             gidx_v, dst_v, msg_v, acc_sh, sem0, sem1, ssem0, ssem1):
    ssems = (ssem0, ssem1)
    c = lax.axis_index("c")
    s = lax.axis_index("s")
    wid = s * N_CORES + c

    pltpu.sync_copy(zeros_hbm.at[pl.ds(s * STRIPE, STRIPE)],
                    acc_sh.at[pl.ds(s * STRIPE, STRIPE)])
    plsc.subcore_barrier()

    row0 = wid * ROWS_PER_TILE
    pltpu.sync_copy(gidx_hbm.at[pl.ds(row0, ROWS_PER_TILE)], gidx_v)
    pltpu.sync_copy(dst_hbm.at[pl.ds(row0, ROWS_PER_TILE)], dst_v)

    gsems = (sem0, sem1)
    gds = {}
    sds = {}

    def gstart(t):
        b0 = (t % 2) * BANK
        gds[t] = [
            pltpu.async_copy(hidden_hbm.at[gidx_v.at[t * BANK + i]],
                             msg_v.at[b0 + i], gsems[t % 2])
            for i in range(BANK)
        ]

    def sstart(t):
        b0 = (t % 2) * BANK
        sds[t] = [
            pltpu.async_copy(msg_v.at[b0 + i],
                             acc_sh.at[dst_v.at[t * BANK + i]],
                             ssems[t % 2], add=True)
            for i in range(BANK)
        ]

    gstart(0)
    for t in range(NBATCH):
        if t + 1 < NBATCH:
            if t >= 1:
                for d in sds[t - 1]:
                    d.wait()
            gstart(t + 1)
        for d in gds[t]:
            d.wait()
        sstart(t)
    for d in sds[NBATCH - 2] + sds[NBATCH - 1]:
        d.wait()

    plsc.subcore_barrier()

    pltpu.sync_copy(acc_sh.at[pl.ds(s * STRIPE, STRIPE)],
                    out_hbm.at[c, pl.ds(s * STRIPE, STRIPE)])


@jax.jit
def kernel(feat, edge_index, etypes, weight):
    src = edge_index[0]
    dst = edge_index[1]
    w2 = weight.transpose(1, 0, 2).reshape(D, R * D)
    src_r = src.reshape(E_ROWS, LANE)
    et_r = etypes.reshape(E_ROWS, LANE)
    dst_r = dst.reshape(E_ROWS, LANE)
    zeros = jnp.zeros((ACC_ROWS, D), jnp.float32)

    hidden = pl.pallas_call(
        _mm_kernel,
        grid=(1, 4),
        in_specs=[
            pl.BlockSpec((N_NODES, D), lambda i, j: (0, 0)),
            pl.BlockSpec((D, LANE), lambda i, j: (0, j)),
        ],
        out_specs=pl.BlockSpec((1, N_NODES, LANE), lambda i, j: (j, 0, 0)),
        out_shape=jax.ShapeDtypeStruct((4, N_NODES, LANE), jnp.float32),
    )(feat, w2)
    hidden = hidden.reshape(N_NODES * R, D)
    gidx2d, dst2d = pl.pallas_call(
        _prep_kernel,
        out_shape=[
            jax.ShapeDtypeStruct((NROWS, LANE), jnp.int32),
            jax.ShapeDtypeStruct((NROWS, LANE), jnp.int32),
        ],
    )(src_r, et_r, dst_r)

    mesh = plsc.VectorSubcoreMesh(core_axis_name="c", subcore_axis_name="s",
                                  num_cores=N_CORES)
    partials = pl.kernel(
        _sc_body,
        out_type=jax.ShapeDtypeStruct((N_CORES, ACC_ROWS, D), jnp.float32),
        mesh=mesh,
        scratch_types=[
            pltpu.VMEM((ROWS_PER_TILE, LANE), jnp.int32),
            pltpu.VMEM((ROWS_PER_TILE, LANE), jnp.int32),
            pltpu.VMEM((2 * BANK, LANE, D), jnp.float32),
            pltpu.VMEM_SHARED((ACC_ROWS, D), jnp.float32),
            pltpu.SemaphoreType.DMA,
            pltpu.SemaphoreType.DMA,
            pltpu.SemaphoreType.DMA,
            pltpu.SemaphoreType.DMA,
        ],
        compiler_params=pltpu.CompilerParams(use_tc_tiling_on_sc=False),
    )(gidx2d, dst2d, hidden, zeros)

    return partials[0, :N_NODES]
